# Initial kernel scaffold; baseline (speedup 1.0000x reference)
import jax
import jax.numpy as jnp
from jax.experimental import pallas as pl


def kernel(x, edge_index, W1, b1, Wmu, bmu, Wls, bls):
    raise NotImplementedError("write your pallas kernel here")


# speedup vs baseline: 15.8766x; 15.8766x over previous
import functools

import jax
import jax.numpy as jnp
from jax import lax
from jax.experimental import pallas as pl
from jax.experimental.pallas import tpu as pltpu
from jax.experimental.pallas import tpu_sc as plsc

N = 10000
E = 320000
C = 128
NC = 2
NS = 16
NW = NC * NS
EW = E // NW
K = 80
NCHUNK = EW // K
ZR = 624
ZR_LAST = N - (NS - 1) * ZR


_deg_mesh = plsc.VectorSubcoreMesh(core_axis_name="c", subcore_axis_name="s")


@functools.partial(
    pl.kernel,
    out_type=jax.ShapeDtypeStruct((NC, 1, N), jnp.float32),
    mesh=_deg_mesh,
    scratch_types=[
        pltpu.VMEM((K,), jnp.int32),
        pltpu.VMEM((K,), jnp.float32),
        pltpu.VMEM((N,), jnp.float32),
        pltpu.VMEM_SHARED((N,), jnp.float32),
    ],
)
def _deg_kernel(dst_hbm, deg_out, idx_v, ones_v, zbuf, acc):
    cid = lax.axis_index("c")
    sid = lax.axis_index("s")
    wid = cid * NS + sid
    base = wid * EW

    for j in range(K // 16):
        ones_v[pl.ds(j * 16, 16)] = jnp.ones((16,), jnp.float32)

    @pl.when(sid == 0)
    def _zero():
        def zrow(i, _):
            zbuf[pl.ds(i * 16, 16)] = jnp.zeros((16,), jnp.float32)
            return 0
        lax.fori_loop(0, N // 16, zrow, 0)
        pltpu.sync_copy(zbuf, acc)

    plsc.subcore_barrier()

    def chunk(i, _):
        off = base + i * K
        pltpu.sync_copy(dst_hbm.at[pl.ds(off, K)], idx_v)
        pltpu.sync_copy(ones_v, acc.at[idx_v], add=True)
        return 0

    lax.fori_loop(0, NCHUNK, chunk, 0)
    plsc.subcore_barrier()

    @pl.when(sid == 0)
    def _out():
        pltpu.sync_copy(acc, deg_out.at[cid, 0])


_prop_mesh = plsc.VectorSubcoreMesh(core_axis_name="c", subcore_axis_name="s")


@functools.partial(
    pl.kernel,
    out_type=jax.ShapeDtypeStruct((NC, N, C), jnp.float32),
    mesh=_prop_mesh,
    scratch_types=[
        pltpu.VMEM((K,), jnp.int32),
        pltpu.VMEM((K,), jnp.int32),
        pltpu.VMEM((K, C), jnp.float32),
        pltpu.VMEM((K, C), jnp.float32),
        pltpu.VMEM_SHARED((N, C), jnp.float32),
        pltpu.SemaphoreType.DMA,
    ],
)
def _prop_kernel(u_hbm, src_hbm, dst_hbm, out_hbm, sidx, didx, rows, zbuf, acc, sem):
    cid = lax.axis_index("c")
    sid = lax.axis_index("s")
    wid = cid * NS + sid
    base = wid * EW

    def zrow(i, _):
        for j in range(C // 16):
            zbuf[i, pl.ds(j * 16, 16)] = jnp.zeros((16,), jnp.float32)
        return 0

    lax.fori_loop(0, K, zrow, 0)
    zb = sid * ZR
    for t in range(ZR_LAST // K):
        pltpu.sync_copy(zbuf, acc.at[pl.ds(zb + t * K, K)])
    plsc.subcore_barrier()

    def chunk(i, _):
        off = base + i * K
        pltpu.sync_copy(src_hbm.at[pl.ds(off, K)], sidx)
        pltpu.sync_copy(dst_hbm.at[pl.ds(off, K)], didx)
        pltpu.async_copy(u_hbm.at[sidx], rows, sem).wait()
        pltpu.sync_copy(rows, acc.at[didx], add=True)
        return 0

    lax.fori_loop(0, NCHUNK, chunk, 0)
    plsc.subcore_barrier()

    @pl.when(sid < NS - 1)
    def _out_body():
        pltpu.sync_copy(acc.at[pl.ds(sid * ZR, ZR)],
                        out_hbm.at[cid, pl.ds(sid * ZR, ZR)])

    @pl.when(sid == NS - 1)
    def _out_last():
        pltpu.sync_copy(acc.at[pl.ds((NS - 1) * ZR, ZR_LAST)],
                        out_hbm.at[cid, pl.ds((NS - 1) * ZR, ZR_LAST)])


BR = 1000


def _dense1_body(x_ref, w_ref, degs_ref, u_ref):
    dinv = lax.rsqrt(degs_ref[...])
    xw = jnp.dot(x_ref[...], w_ref[...], preferred_element_type=jnp.float32)
    u_ref[...] = xw * dinv


def _dense2_body(s_ref, u_ref, degs_ref, b_ref, u2_ref):
    dinv = lax.rsqrt(degs_ref[...])
    pre = dinv * (s_ref[0] + s_ref[1] + u_ref[...]) + b_ref[...]
    u2_ref[...] = dinv * jnp.maximum(pre, 0.0)


def _dense3_body(s_ref, u_ref, degs_ref, wmu_ref, bmu_ref, wls_ref, bls_ref,
                 mu_ref, ls_ref):
    dinv = lax.rsqrt(degs_ref[...])
    g = dinv * (s_ref[0] + s_ref[1] + u_ref[...])
    mu_ref[...] = jnp.dot(g, wmu_ref[...],
                          preferred_element_type=jnp.float32) + bmu_ref[...]
    ls_ref[...] = jnp.dot(g, wls_ref[...],
                          preferred_element_type=jnp.float32) + bls_ref[...]


def _rows(i):
    return (i, 0)


def kernel(x, edge_index, W1, b1, Wmu, bmu, Wls, bls):
    ei = edge_index.astype(jnp.int32)
    src, dst = ei[0], ei[1]

    deg2 = _deg_kernel(dst)
    degs = (deg2[0, 0] + deg2[1, 0] + 1.0).reshape(N, 1)

    grid = N // BR
    u1 = pl.pallas_call(
        _dense1_body,
        grid=(grid,),
        in_specs=[
            pl.BlockSpec((BR, C), _rows),
            pl.BlockSpec((C, C), lambda i: (0, 0)),
            pl.BlockSpec((BR, 1), _rows),
        ],
        out_specs=pl.BlockSpec((BR, C), _rows),
        out_shape=jax.ShapeDtypeStruct((N, C), jnp.float32),
    )(x, W1, degs)

    s1 = _prop_kernel(u1, src, dst)

    u2 = pl.pallas_call(
        _dense2_body,
        grid=(grid,),
        in_specs=[
            pl.BlockSpec((2, BR, C), lambda i: (0, i, 0)),
            pl.BlockSpec((BR, C), _rows),
            pl.BlockSpec((BR, 1), _rows),
            pl.BlockSpec((1, C), lambda i: (0, 0)),
        ],
        out_specs=pl.BlockSpec((BR, C), _rows),
        out_shape=jax.ShapeDtypeStruct((N, C), jnp.float32),
    )(s1, u1, degs, b1.reshape(1, C))

    s2 = _prop_kernel(u2, src, dst)

    OC = Wmu.shape[1]
    mu, ls = pl.pallas_call(
        _dense3_body,
        grid=(grid,),
        in_specs=[
            pl.BlockSpec((2, BR, C), lambda i: (0, i, 0)),
            pl.BlockSpec((BR, C), _rows),
            pl.BlockSpec((BR, 1), _rows),
            pl.BlockSpec((C, OC), lambda i: (0, 0)),
            pl.BlockSpec((1, OC), lambda i: (0, 0)),
            pl.BlockSpec((C, OC), lambda i: (0, 0)),
            pl.BlockSpec((1, OC), lambda i: (0, 0)),
        ],
        out_specs=[pl.BlockSpec((BR, OC), _rows), pl.BlockSpec((BR, OC), _rows)],
        out_shape=[jax.ShapeDtypeStruct((N, OC), jnp.float32),
                   jax.ShapeDtypeStruct((N, OC), jnp.float32)],
    )(s2, u2, degs, Wmu, bmu.reshape(1, OC), Wls, bls.reshape(1, OC))

    return (mu, ls)


# speedup vs baseline: 36.9601x; 2.3280x over previous
import functools

import jax
import jax.numpy as jnp
from jax import lax
from jax.experimental import pallas as pl
from jax.experimental.pallas import tpu as pltpu
from jax.experimental.pallas import tpu_sc as plsc

N = 10000
E = 320000
C = 128
NC = 2
NS = 16
NW = NC * NS
EW = E // NW
K = 80
NCHUNK = EW // K
ZR = 624
ZR_LAST = N - (NS - 1) * ZR


_deg_mesh = plsc.VectorSubcoreMesh(core_axis_name="c", subcore_axis_name="s")


@functools.partial(
    pl.kernel,
    out_type=jax.ShapeDtypeStruct((NC, 1, N), jnp.float32),
    mesh=_deg_mesh,
    scratch_types=[
        pltpu.VMEM((NCHUNK, K), jnp.int32),
        pltpu.VMEM((K,), jnp.float32),
        pltpu.VMEM((N,), jnp.float32),
        pltpu.VMEM_SHARED((N,), jnp.float32),
        pltpu.SemaphoreType.DMA,
    ],
)
def _deg_kernel(dst3_hbm, deg_out, idx_v, ones_v, zbuf, acc, sem):
    cid = lax.axis_index("c")
    sid = lax.axis_index("s")
    wid = cid * NS + sid

    cp = pltpu.async_copy(dst3_hbm.at[wid], idx_v, sem)

    for j in range(K // 16):
        ones_v[pl.ds(j * 16, 16)] = jnp.ones((16,), jnp.float32)

    @pl.when(sid == 0)
    def _zero():
        def zrow(i, _):
            zbuf[pl.ds(i * 16, 16)] = jnp.zeros((16,), jnp.float32)
            return 0
        lax.fori_loop(0, N // 16, zrow, 0)
        pltpu.sync_copy(zbuf, acc)

    cp.wait()
    plsc.subcore_barrier()

    def chunk(i, _):
        pltpu.sync_copy(ones_v, acc.at[idx_v.at[i]], add=True)
        return 0

    lax.fori_loop(0, NCHUNK, chunk, 0)
    plsc.subcore_barrier()

    @pl.when(sid == 0)
    def _out():
        pltpu.sync_copy(acc, deg_out.at[cid, 0])


_prop_mesh = plsc.VectorSubcoreMesh(core_axis_name="c", subcore_axis_name="s")


@functools.partial(
    pl.kernel,
    out_type=jax.ShapeDtypeStruct((NC, N, C), jnp.float32),
    mesh=_prop_mesh,
    scratch_types=[
        pltpu.VMEM((EW,), jnp.int32),
        pltpu.VMEM((NCHUNK, K), jnp.int32),
        pltpu.VMEM((K, C), jnp.float32),
        pltpu.VMEM((K, C), jnp.float32),
        pltpu.VMEM_SHARED((N, C), jnp.float32),
        pltpu.SemaphoreType.DMA,
        pltpu.SemaphoreType.DMA,
        pltpu.SemaphoreType.DMA,
    ],
)
def _prop_kernel(u_hbm, src_hbm, dst3_hbm, out_hbm, sidx, didx, rows0, rows1,
                 acc, semi, sem0, sem1):
    cid = lax.axis_index("c")
    sid = lax.axis_index("s")
    wid = cid * NS + sid

    cps = pltpu.async_copy(src_hbm.at[pl.ds(wid * EW, EW)], sidx, semi)
    cpd = pltpu.async_copy(dst3_hbm.at[wid], didx, semi)

    def zrow(i, _):
        for j in range(C // 16):
            rows0[i, pl.ds(j * 16, 16)] = jnp.zeros((16,), jnp.float32)
            rows1[i, pl.ds(j * 16, 16)] = jnp.zeros((16,), jnp.float32)
        return 0

    lax.fori_loop(0, K, zrow, 0)
    zb = sid * ZR
    for t in range(ZR_LAST // (2 * K)):
        pltpu.sync_copy(rows0, acc.at[pl.ds(zb + 2 * t * K, K)])
        pltpu.sync_copy(rows1, acc.at[pl.ds(zb + (2 * t + 1) * K, K)])
    cps.wait()
    cpd.wait()
    plsc.subcore_barrier()

    pltpu.async_copy(u_hbm.at[sidx.at[pl.ds(0, K)]], rows0, sem0)

    def pair(j, _):
        i = j * 2
        pltpu.async_copy(u_hbm.at[sidx.at[pl.ds((i + 1) * K, K)]], rows1, sem1)
        pltpu.make_async_copy(u_hbm.at[sidx.at[pl.ds(i * K, K)]], rows0,
                              sem0).wait()
        pltpu.sync_copy(rows0, acc.at[didx.at[i]], add=True)
        pltpu.async_copy(u_hbm.at[sidx.at[pl.ds((i + 2) * K, K)]], rows0, sem0)
        pltpu.make_async_copy(u_hbm.at[sidx.at[pl.ds((i + 1) * K, K)]], rows1,
                              sem1).wait()
        pltpu.sync_copy(rows1, acc.at[didx.at[i + 1]], add=True)
        return 0

    lax.fori_loop(0, (NCHUNK - 1) // 2, pair, 0)
    pltpu.make_async_copy(u_hbm.at[sidx.at[pl.ds((NCHUNK - 1) * K, K)]], rows0,
                          sem0).wait()
    pltpu.sync_copy(rows0, acc.at[didx.at[NCHUNK - 1]], add=True)
    plsc.subcore_barrier()

    @pl.when(sid < NS - 1)
    def _out_body():
        pltpu.sync_copy(acc.at[pl.ds(sid * ZR, ZR)],
                        out_hbm.at[cid, pl.ds(sid * ZR, ZR)])

    @pl.when(sid == NS - 1)
    def _out_last():
        pltpu.sync_copy(acc.at[pl.ds((NS - 1) * ZR, ZR_LAST)],
                        out_hbm.at[cid, pl.ds((NS - 1) * ZR, ZR_LAST)])


BR = 1000


def _dense1_body(x_ref, w_ref, degs_ref, u_ref):
    dinv = lax.rsqrt(degs_ref[...])
    xw = jnp.dot(x_ref[...], w_ref[...], preferred_element_type=jnp.float32)
    u_ref[...] = xw * dinv


def _dense2_body(s_ref, u_ref, degs_ref, b_ref, u2_ref):
    dinv = lax.rsqrt(degs_ref[...])
    pre = dinv * (s_ref[0] + s_ref[1] + u_ref[...]) + b_ref[...]
    u2_ref[...] = dinv * jnp.maximum(pre, 0.0)


def _dense3_body(s_ref, u_ref, degs_ref, wmu_ref, bmu_ref, wls_ref, bls_ref,
                 mu_ref, ls_ref):
    dinv = lax.rsqrt(degs_ref[...])
    g = dinv * (s_ref[0] + s_ref[1] + u_ref[...])
    mu_ref[...] = jnp.dot(g, wmu_ref[...],
                          preferred_element_type=jnp.float32) + bmu_ref[...]
    ls_ref[...] = jnp.dot(g, wls_ref[...],
                          preferred_element_type=jnp.float32) + bls_ref[...]


def _rows(i):
    return (i, 0)


def kernel(x, edge_index, W1, b1, Wmu, bmu, Wls, bls):
    ei = edge_index.astype(jnp.int32)
    src = ei[0]
    dst3 = ei[1].reshape(NW, NCHUNK, K)

    deg2 = _deg_kernel(dst3)
    degs = (deg2[0, 0] + deg2[1, 0] + 1.0).reshape(N, 1)

    grid = N // BR
    u1 = pl.pallas_call(
        _dense1_body,
        grid=(grid,),
        in_specs=[
            pl.BlockSpec((BR, C), _rows),
            pl.BlockSpec((C, C), lambda i: (0, 0)),
            pl.BlockSpec((BR, 1), _rows),
        ],
        out_specs=pl.BlockSpec((BR, C), _rows),
        out_shape=jax.ShapeDtypeStruct((N, C), jnp.float32),
    )(x, W1, degs)

    s1 = _prop_kernel(u1, src, dst3)

    u2 = pl.pallas_call(
        _dense2_body,
        grid=(grid,),
        in_specs=[
            pl.BlockSpec((2, BR, C), lambda i: (0, i, 0)),
            pl.BlockSpec((BR, C), _rows),
            pl.BlockSpec((BR, 1), _rows),
            pl.BlockSpec((1, C), lambda i: (0, 0)),
        ],
        out_specs=pl.BlockSpec((BR, C), _rows),
        out_shape=jax.ShapeDtypeStruct((N, C), jnp.float32),
    )(s1, u1, degs, b1.reshape(1, C))

    s2 = _prop_kernel(u2, src, dst3)

    OC = Wmu.shape[1]
    mu, ls = pl.pallas_call(
        _dense3_body,
        grid=(grid,),
        in_specs=[
            pl.BlockSpec((2, BR, C), lambda i: (0, i, 0)),
            pl.BlockSpec((BR, C), _rows),
            pl.BlockSpec((BR, 1), _rows),
            pl.BlockSpec((C, OC), lambda i: (0, 0)),
            pl.BlockSpec((1, OC), lambda i: (0, 0)),
            pl.BlockSpec((C, OC), lambda i: (0, 0)),
            pl.BlockSpec((1, OC), lambda i: (0, 0)),
        ],
        out_specs=[pl.BlockSpec((BR, OC), _rows), pl.BlockSpec((BR, OC), _rows)],
        out_shape=[jax.ShapeDtypeStruct((N, OC), jnp.float32),
                   jax.ShapeDtypeStruct((N, OC), jnp.float32)],
    )(s2, u2, degs, Wmu, bmu.reshape(1, OC), Wls, bls.reshape(1, OC))

    return (mu, ls)
